# baseline (speedup 1.0000x reference)
import functools

import jax
import jax.numpy as jnp
from jax import lax
from jax.experimental import pallas as pl
from jax.experimental.pallas import tpu as pltpu
from jax.experimental.pallas import tpu_sc as plsc

N_DEPTHS = 3
N_LABELS = 1000
BATCH = 16384
D = 1000

NC = 2
NS = 16
NW = NC * NS
G = 8
NBAND = N_DEPTHS * (N_LABELS // G)
TBW = G * N_LABELS
QI = 4096
NQ = BATCH // QI
NPJ = (NBAND + 2 * NW - 1) // (2 * NW)
LANES = 16


def _sc_transposed_gather(leaf, advT_flat):
    mesh = plsc.VectorSubcoreMesh(core_axis_name="c", subcore_axis_name="s")

    @functools.partial(
        pl.kernel,
        mesh=mesh,
        out_type=jax.ShapeDtypeStruct((N_DEPTHS, N_LABELS, BATCH), jnp.float32),
        compiler_params=pltpu.CompilerParams(needs_layout_passes=False),
        scratch_types=[
            pltpu.VMEM((BATCH,), jnp.int32),
            pltpu.VMEM((TBW,), jnp.float32),
            pltpu.VMEM((TBW,), jnp.float32),
            pltpu.VMEM((G, QI), jnp.float32),
            pltpu.VMEM((G, QI), jnp.float32),
            pltpu.SemaphoreType.DMA,
            pltpu.SemaphoreType.DMA,
            pltpu.SemaphoreType.DMA,
            pltpu.SemaphoreType.DMA,
        ],
    )
    def run(leaf_hbm, advT_hbm, out_hbm, leaf_v, tbl0, tbl1, qb0, qb1,
            tg0, tg1, qp0, qp1):
        cid = lax.axis_index("c")
        sid = lax.axis_index("s")
        wid = sid * NC + cid

        pltpu.sync_copy(leaf_hbm, leaf_v)

        tbls = (tbl0, tbl1)
        tsems = (tg0, tg1)
        qbufs = (qb0, qb1)
        qsems = (qp0, qp1)

        def start_tbl(t, b):
            pltpu.async_copy(
                advT_hbm.at[pl.ds(t * TBW, TBW)], tbls[b], tsems[b])

        def wait_tbl(b):
            pltpu.make_async_copy(
                advT_hbm.at[pl.ds(0, TBW)], tbls[b], tsems[b]).wait()

        def wait_qput(b):
            pltpu.make_async_copy(
                qbufs[b],
                out_hbm.at[0, pl.ds(0, G), pl.ds(0, QI)], qsems[b]).wait()

        def compute_quarter(q, tb, qb):
            @plsc.parallel_loop(0, QI // 128, unroll=4)
            def ig_body(ig):
                col0 = ig * 128
                for r in range(128 // LANES):
                    c = col0 + r * LANES
                    idx = leaf_v[pl.ds(q * QI + c, LANES)]
                    for jj in range(G):
                        v = plsc.load_gather(
                            tbls[tb].at[pl.ds(jj * N_LABELS, N_LABELS)],
                            [idx])
                        qbufs[qb][jj, pl.ds(c, LANES)] = v

        def band(j, t, tb, first_pair):
            d = t // (N_LABELS // G)
            jg = t % (N_LABELS // G)
            wait_tbl(tb)
            for q in range(NQ):
                qb = q % 2
                if first_pair and q < 2:
                    @pl.when(j > 0)
                    def _():
                        wait_qput(qb)
                else:
                    wait_qput(qb)
                compute_quarter(q, tb, qb)
                pltpu.async_copy(
                    qbufs[qb],
                    out_hbm.at[d, pl.ds(jg * G, G), pl.ds(q * QI, QI)],
                    qsems[qb])

        start_tbl(wid, 0)
        start_tbl(wid + NW, 1)

        def pair_body(j, carry):
            tA = wid + 2 * NW * j
            tB = tA + NW
            tA_next = tA + 2 * NW
            tB_next = tB + 2 * NW

            @pl.when(tA < NBAND)
            def _():
                band(j, tA, 0, True)

            @pl.when(tA_next < NBAND)
            def _():
                start_tbl(tA_next, 0)

            @pl.when(tB < NBAND)
            def _():
                band(j, tB, 1, False)

            @pl.when(tB_next < NBAND)
            def _():
                start_tbl(tB_next, 1)

            return carry

        lax.fori_loop(0, NPJ, pair_body, 0)

        wait_qput(0)
        wait_qput(1)

    return run(leaf, advT_flat)


def kernel(labels, adversaries):
    leaf = labels[:, -1]
    advT = jnp.swapaxes(adversaries, 1, 2).reshape(N_DEPTHS * N_LABELS * D)
    out_t = _sc_transposed_gather(leaf, advT)
    return jnp.swapaxes(out_t, 1, 2)


# speedup vs baseline: 1.0698x; 1.0698x over previous
import functools

import jax
import jax.numpy as jnp
from jax import lax
from jax.experimental import pallas as pl
from jax.experimental.pallas import tpu as pltpu
from jax.experimental.pallas import tpu_sc as plsc

N_DEPTHS = 3
N_LABELS = 1000
BATCH = 16384
D = 1000

NC = 2
NS = 16
NW = NC * NS
G = 8
NBAND = N_DEPTHS * (N_LABELS // G)
TBW = G * N_LABELS
QI = 4096
NQ = BATCH // QI
NPJ = (NBAND + 2 * NW - 1) // (2 * NW)
LANES = 16


def _sc_transposed_gather(leaf, advT_flat):
    mesh = plsc.VectorSubcoreMesh(core_axis_name="c", subcore_axis_name="s")

    @functools.partial(
        pl.kernel,
        mesh=mesh,
        out_type=jax.ShapeDtypeStruct((N_DEPTHS, N_LABELS, BATCH), jnp.float32),
        compiler_params=pltpu.CompilerParams(needs_layout_passes=False),
        scratch_types=[
            pltpu.VMEM((BATCH,), jnp.int32),
            pltpu.VMEM((TBW,), jnp.float32),
            pltpu.VMEM((TBW,), jnp.float32),
            pltpu.VMEM((G, QI), jnp.float32),
            pltpu.VMEM((G, QI), jnp.float32),
            pltpu.SemaphoreType.DMA,
            pltpu.SemaphoreType.DMA,
            pltpu.SemaphoreType.DMA,
            pltpu.SemaphoreType.DMA,
        ],
    )
    def run(leaf_hbm, advT_hbm, out_hbm, leaf_v, tbl0, tbl1, qb0, qb1,
            tg0, tg1, qp0, qp1):
        cid = lax.axis_index("c")
        sid = lax.axis_index("s")
        wid = sid * NC + cid

        pltpu.sync_copy(leaf_hbm, leaf_v)

        tbls = (tbl0, tbl1)
        tsems = (tg0, tg1)
        qbufs = (qb0, qb1)
        qsems = (qp0, qp1)

        def start_tbl(t, b):
            pltpu.async_copy(
                advT_hbm.at[pl.ds(t * TBW, TBW)], tbls[b], tsems[b])

        def wait_tbl(b):
            pltpu.make_async_copy(
                advT_hbm.at[pl.ds(0, TBW)], tbls[b], tsems[b]).wait()

        def wait_qput(b):
            pltpu.make_async_copy(
                qbufs[b],
                out_hbm.at[0, pl.ds(0, G), pl.ds(0, QI)], qsems[b]).wait()

        def compute_quarter(q, tb, qb):
            @plsc.parallel_loop(0, QI // 128, unroll=2)
            def ig_body(ig):
                col0 = ig * 128
                for r in range(128 // LANES):
                    c = col0 + r * LANES
                    idx = leaf_v[pl.ds(q * QI + c, LANES)]
                    for jj in range(G):
                        v = plsc.load_gather(
                            tbls[tb].at[pl.ds(jj * N_LABELS, N_LABELS)],
                            [idx])
                        qbufs[qb][jj, pl.ds(c, LANES)] = v

        def band(j, t, tb, first_pair):
            d = t // (N_LABELS // G)
            jg = t % (N_LABELS // G)
            wait_tbl(tb)
            for q in range(NQ):
                qb = q % 2
                if first_pair and q < 2:
                    @pl.when(j > 0)
                    def _():
                        wait_qput(qb)
                else:
                    wait_qput(qb)
                compute_quarter(q, tb, qb)
                pltpu.async_copy(
                    qbufs[qb],
                    out_hbm.at[d, pl.ds(jg * G, G), pl.ds(q * QI, QI)],
                    qsems[qb])

        start_tbl(wid, 0)
        start_tbl(wid + NW, 1)

        def pair_body(j, carry):
            tA = wid + 2 * NW * j
            tB = tA + NW
            tA_next = tA + 2 * NW
            tB_next = tB + 2 * NW

            @pl.when(tA < NBAND)
            def _():
                band(j, tA, 0, True)

            @pl.when(tA_next < NBAND)
            def _():
                start_tbl(tA_next, 0)

            @pl.when(tB < NBAND)
            def _():
                band(j, tB, 1, False)

            @pl.when(tB_next < NBAND)
            def _():
                start_tbl(tB_next, 1)

            return carry

        lax.fori_loop(0, NPJ, pair_body, 0)

        wait_qput(0)
        wait_qput(1)

    return run(leaf, advT_flat)


def kernel(labels, adversaries):
    leaf = labels[:, -1]
    advT = jnp.swapaxes(adversaries, 1, 2).reshape(N_DEPTHS * N_LABELS * D)
    out_t = _sc_transposed_gather(leaf, advT)
    return jnp.swapaxes(out_t, 1, 2)
